# baseline (speedup 1.0000x reference)
import functools

import jax
import jax.numpy as jnp
from jax import lax
from jax.experimental import pallas as pl
from jax.experimental.pallas import tpu as pltpu
from jax.experimental.pallas import tpu_sc as plsc

V = 100000
D = 64
H = 128
C0 = 6667
C1 = 20001
S = 50
B = 32
HEAD = C0 + 2
N = S * B

_SC_ROWS = 64
_SC_WORKERS = N // _SC_ROWS


def _row_gather(tbl, idx, width, label):
    mesh = plsc.VectorSubcoreMesh(core_axis_name="c", subcore_axis_name="s")

    @functools.partial(
        pl.kernel,
        mesh=mesh,
        out_type=jax.ShapeDtypeStruct((N * width,), jnp.float32),
        scratch_types=[
            pltpu.VMEM((_SC_ROWS,), jnp.int32),
            pltpu.VMEM((_SC_ROWS * width,), jnp.float32),
            pltpu.SemaphoreType.DMA,
        ],
        name=label,
        compiler_params=pltpu.CompilerParams(needs_layout_passes=False),
    )
    def gather_kernel(tbl_h, idx_h, out_o, idx_v, rows_v, sem):
        wid = lax.axis_index("s") * 2 + lax.axis_index("c")

        @pl.when(wid < _SC_WORKERS)
        def _():
            base = wid * _SC_ROWS
            pltpu.sync_copy(idx_h.at[pl.ds(base, _SC_ROWS)], idx_v)
            lane = lax.broadcasted_iota(jnp.int32, (16,), 0)
            for g in range(_SC_ROWS // 16):
                grp = idx_v[pl.ds(g * 16, 16)]
                for l in range(16):
                    r = jnp.sum(jnp.where(lane == l, grp, 0))
                    j = g * 16 + l
                    pltpu.async_copy(tbl_h.at[pl.ds(r * width, width)],
                                     rows_v.at[pl.ds(j * width, width)], sem)
            pltpu.make_async_copy(tbl_h.at[pl.ds(0, _SC_ROWS * width)],
                                  rows_v, sem).wait()
            pltpu.sync_copy(rows_v, out_o.at[pl.ds(base * width,
                                                   _SC_ROWS * width)])

    return gather_kernel(tbl.reshape(-1), idx)


def _gather_head(head_W, idx_h):
    mesh = plsc.VectorSubcoreMesh(core_axis_name="c", subcore_axis_name="s")

    @functools.partial(
        pl.kernel,
        mesh=mesh,
        out_type=jax.ShapeDtypeStruct((N, H), jnp.float32),
        scratch_types=[
            pltpu.VMEM((_SC_ROWS,), jnp.int32),
            pltpu.VMEM((_SC_ROWS, H), jnp.float32),
            pltpu.SemaphoreType.DMA,
        ],
    )
    def gather_kernel(hw_h, ih_h, wh_o, ih_v, wh_v, sem):
        wid = lax.axis_index("s") * 2 + lax.axis_index("c")

        @pl.when(wid < _SC_WORKERS)
        def _():
            base = wid * _SC_ROWS
            pltpu.sync_copy(ih_h.at[pl.ds(base, _SC_ROWS)], ih_v)
            pltpu.async_copy(hw_h.at[ih_v], wh_v, sem).wait()
            pltpu.sync_copy(wh_v, wh_o.at[pl.ds(base, _SC_ROWS)])

    return gather_kernel(head_W, idx_h)


def _lstm_body(x_ref, wih_ref, whh_ref, b_ref, p0_ref, p1_ref,
               h2_ref, flat_ref, pr0_ref, pr1_ref, hd2_ref, xp_ref):
    x = x_ref[...]
    xp = lax.dot_general(
        x.astype(jnp.bfloat16), wih_ref[...].astype(jnp.bfloat16),
        (((1,), (1,)), ((), ())),
        preferred_element_type=jnp.float32) + b_ref[...]
    xp_ref[...] = xp.reshape(S, B, 4 * H)
    whh_b = whh_ref[...].astype(jnp.bfloat16)

    def step(t, carry):
        h, c = carry
        z = xp_ref[t] + lax.dot_general(
            h.astype(jnp.bfloat16), whh_b, (((1,), (1,)), ((), ())),
            preferred_element_type=jnp.float32)
        i = jax.nn.sigmoid(z[:, :H])
        f = jax.nn.sigmoid(z[:, H:2 * H])
        g = jnp.tanh(z[:, 2 * H:3 * H])
        o = jax.nn.sigmoid(z[:, 3 * H:])
        c = f * c + i * g
        h = o * jnp.tanh(c)
        flat_ref[t] = h
        return (h, c)

    lax.fori_loop(0, S, step,
                  (jnp.zeros((B, H), jnp.float32),
                   jnp.zeros((B, H), jnp.float32)),
                  unroll=2)
    flat = flat_ref[...].reshape(N, H)
    pr0_ref[...] = lax.dot_general(flat, p0_ref[...], (((1,), (1,)), ((), ())),
                                   preferred_element_type=jnp.float32)
    pr1_ref[...] = lax.dot_general(flat, p1_ref[...], (((1,), (1,)), ((), ())),
                                   preferred_element_type=jnp.float32)
    hd2_ref[...] = lax.dot_general(flat, h2_ref[...], (((1,), (1,)), ((), ())),
                                   preferred_element_type=jnp.float32)


def _lstm(x, W_ih, W_hh, b2, tail0_proj, tail1_proj, headW2):
    return pl.pallas_call(
        _lstm_body,
        out_shape=[
            jax.ShapeDtypeStruct((S, B, H), jnp.float32),
            jax.ShapeDtypeStruct((N, H // 2), jnp.float32),
            jax.ShapeDtypeStruct((N, H // 4), jnp.float32),
            jax.ShapeDtypeStruct((N, 2), jnp.float32),
        ],
        scratch_shapes=[pltpu.VMEM((S, B, 4 * H), jnp.float32)],
    )(x, W_ih, W_hh, b2, tail0_proj, tail1_proj, headW2)


def _lse_sum(proj, w, rows, chunk):
    grid = -(-rows // chunk)
    last = rows - (grid - 1) * chunk
    k_dim = proj.shape[1]

    def body(p_ref, w_ref, s_ref):
        i = pl.program_id(0)

        @pl.when(i == 0)
        def _():
            s_ref[...] = jnp.zeros_like(s_ref)

        logits = lax.dot_general(
            p_ref[...].astype(jnp.bfloat16), w_ref[...].astype(jnp.bfloat16),
            (((1,), (1,)), ((), ())),
            preferred_element_type=jnp.float32)
        e = jnp.exp(logits)
        if last == chunk:
            s_ref[...] += jnp.sum(e, axis=1, keepdims=True)
        else:
            @pl.when(i < grid - 1)
            def _():
                s_ref[...] += jnp.sum(e, axis=1, keepdims=True)

            @pl.when(i == grid - 1)
            def _():
                col = lax.broadcasted_iota(jnp.int32, e.shape, 1)
                s_ref[...] += jnp.sum(jnp.where(col < last, e, 0.0),
                                      axis=1, keepdims=True)

    return pl.pallas_call(
        body,
        grid=(grid,),
        in_specs=[
            pl.BlockSpec((N, k_dim), lambda i: (0, 0)),
            pl.BlockSpec((chunk, k_dim), lambda i: (i, 0)),
        ],
        out_specs=pl.BlockSpec((N, 1), lambda i: (0, 0)),
        out_shape=jax.ShapeDtypeStruct((N, 1), jnp.float32),
        compiler_params=pltpu.CompilerParams(
            dimension_semantics=("arbitrary",)),
    )(proj, w)


def _assemble_body(tgt_ref, sh_ref, s0_ref, s1_ref, hd2_ref, fl_ref, wh_ref,
                   p0_ref, w0_ref, p1_ref, w1_ref, o_ref):
    tgt = tgt_ref[...]
    lse_h = jnp.log(sh_ref[...])
    lse0 = jnp.log(s0_ref[...])
    lse1 = jnp.log(s1_ref[...])
    th = jnp.sum(fl_ref[...] * wh_ref[...], axis=1, keepdims=True)
    t0 = jnp.sum(p0_ref[...] * w0_ref[...], axis=1, keepdims=True)
    t1 = jnp.sum(p1_ref[...] * w1_ref[...], axis=1, keepdims=True)
    hd2 = hd2_ref[...]
    out = jnp.where(tgt < C0, th - lse_h, 0.0)
    out = jnp.where((tgt >= C0) & (tgt < C1),
                    hd2[:, 0:1] - lse_h + t0 - lse0, out)
    out = jnp.where(tgt >= C1, hd2[:, 1:2] - lse_h + t1 - lse1, out)
    o_ref[...] = jnp.full((1, 1), -1.0 / N, jnp.float32) * jnp.sum(out)


def _assemble(tgt2, s_h, s_0, s_1, hd2, flat, wh, pr0, w0, pr1, w1):
    return pl.pallas_call(
        _assemble_body,
        out_shape=jax.ShapeDtypeStruct((1, 1), jnp.float32),
    )(tgt2, s_h, s_0, s_1, hd2, flat, wh, pr0, w0, pr1, w1)


def kernel(review_input, review_output, emb, W_ih, W_hh, b_ih, b_hh,
           head_W, tail0_proj, tail0_out, tail1_proj, tail1_out):
    ie = review_input.reshape(-1).astype(jnp.int32)
    tgt = review_output.reshape(-1).astype(jnp.int32)
    ih = jnp.clip(tgt, 0, C0 - 1)
    i0 = jnp.clip(tgt - C0, 0, C1 - C0 - 1)
    i1 = jnp.clip(tgt - C1, 0, V - C1 - 1)

    x = _row_gather(emb, ie, D, "emb_row_gather").reshape(N, D)
    wh = _gather_head(head_W, ih)
    w0 = _row_gather(tail0_out, i0, H // 2,
                     "tail0_row_gather").reshape(N, H // 2)
    w1 = _row_gather(tail1_out, i1, H // 4,
                     "tail1_row_gather").reshape(N, H // 4)

    b2 = (b_ih + b_hh).reshape(1, 4 * H)
    headW2 = lax.slice(head_W, (C0, 0), (C0 + 2, H))
    flat3, pr0, pr1, hd2 = _lstm(x, W_ih, W_hh, b2, tail0_proj,
                                 tail1_proj, headW2)
    flat = flat3.reshape(N, H)

    s_h = _lse_sum(flat, head_W, HEAD, 2048)
    s_0 = _lse_sum(pr0, tail0_out, C1 - C0, 2048)
    s_1 = _lse_sum(pr1, tail1_out, V - C1, 2048)

    loss = _assemble(tgt.reshape(N, 1), s_h, s_0, s_1, hd2, flat, wh,
                     pr0, w0, pr1, w1)
    return loss.reshape(())


# speedup vs baseline: 1.0761x; 1.0761x over previous
import functools

import jax
import jax.numpy as jnp
from jax import lax
from jax.experimental import pallas as pl
from jax.experimental.pallas import tpu as pltpu
from jax.experimental.pallas import tpu_sc as plsc

V = 100000
D = 64
H = 128
C0 = 6667
C1 = 20001
S = 50
B = 32
HEAD = C0 + 2
N = S * B

_SC_ROWS = 64
_SC_WORKERS = N // _SC_ROWS


def _row_gather(tbl, idx, width, label):
    mesh = plsc.VectorSubcoreMesh(core_axis_name="c", subcore_axis_name="s")

    @functools.partial(
        pl.kernel,
        mesh=mesh,
        out_type=jax.ShapeDtypeStruct((N, width), jnp.float32),
        scratch_types=[
            pltpu.VMEM((_SC_ROWS,), jnp.int32),
            pltpu.VMEM((_SC_ROWS, width), jnp.float32),
            pltpu.SemaphoreType.DMA,
        ],
        name=label,
        compiler_params=pltpu.CompilerParams(needs_layout_passes=False),
    )
    def gather_kernel(tbl_h, idx_h, out_o, idx_v, rows_v, sem):
        wid = lax.axis_index("s") * 2 + lax.axis_index("c")

        @pl.when(wid < _SC_WORKERS)
        def _():
            base = wid * _SC_ROWS
            pltpu.sync_copy(idx_h.at[pl.ds(base, _SC_ROWS)], idx_v)
            lane = lax.broadcasted_iota(jnp.int32, (16,), 0)
            for g in range(_SC_ROWS // 16):
                grp = idx_v[pl.ds(g * 16, 16)]
                for l in range(16):
                    r = jnp.sum(jnp.where(lane == l, grp, 0))
                    pltpu.async_copy(tbl_h.at[pl.ds(r, 1), :],
                                     rows_v.at[pl.ds(g * 16 + l, 1), :], sem)
            pltpu.make_async_copy(tbl_h.at[pl.ds(0, _SC_ROWS), :],
                                  rows_v, sem).wait()
            pltpu.sync_copy(rows_v, out_o.at[pl.ds(base, _SC_ROWS)])

    return gather_kernel(tbl, idx)


def _gather_head(head_W, idx_h):
    mesh = plsc.VectorSubcoreMesh(core_axis_name="c", subcore_axis_name="s")

    @functools.partial(
        pl.kernel,
        mesh=mesh,
        out_type=jax.ShapeDtypeStruct((N, H), jnp.float32),
        scratch_types=[
            pltpu.VMEM((_SC_ROWS,), jnp.int32),
            pltpu.VMEM((_SC_ROWS, H), jnp.float32),
            pltpu.SemaphoreType.DMA,
        ],
    )
    def gather_kernel(hw_h, ih_h, wh_o, ih_v, wh_v, sem):
        wid = lax.axis_index("s") * 2 + lax.axis_index("c")

        @pl.when(wid < _SC_WORKERS)
        def _():
            base = wid * _SC_ROWS
            pltpu.sync_copy(ih_h.at[pl.ds(base, _SC_ROWS)], ih_v)
            pltpu.async_copy(hw_h.at[ih_v], wh_v, sem).wait()
            pltpu.sync_copy(wh_v, wh_o.at[pl.ds(base, _SC_ROWS)])

    return gather_kernel(head_W, idx_h)


def _lstm_body(x_ref, wih_ref, whh_ref, b_ref, p0_ref, p1_ref,
               h2_ref, flat_ref, pr0_ref, pr1_ref, hd2_ref, xp_ref):
    x = x_ref[...]
    xp = lax.dot_general(
        x.astype(jnp.bfloat16), wih_ref[...].astype(jnp.bfloat16),
        (((1,), (1,)), ((), ())),
        preferred_element_type=jnp.float32) + b_ref[...]
    xp_ref[...] = xp.reshape(S, B, 4 * H)
    whh_b = whh_ref[...].astype(jnp.bfloat16)

    def step(t, carry):
        h, c = carry
        z = xp_ref[t] + lax.dot_general(
            h.astype(jnp.bfloat16), whh_b, (((1,), (1,)), ((), ())),
            preferred_element_type=jnp.float32)
        i = jax.nn.sigmoid(z[:, :H])
        f = jax.nn.sigmoid(z[:, H:2 * H])
        g = jnp.tanh(z[:, 2 * H:3 * H])
        o = jax.nn.sigmoid(z[:, 3 * H:])
        c = f * c + i * g
        h = o * jnp.tanh(c)
        flat_ref[t] = h
        return (h, c)

    lax.fori_loop(0, S, step,
                  (jnp.zeros((B, H), jnp.float32),
                   jnp.zeros((B, H), jnp.float32)),
                  unroll=2)
    flat = flat_ref[...].reshape(N, H)
    pr0_ref[...] = lax.dot_general(flat, p0_ref[...], (((1,), (1,)), ((), ())),
                                   preferred_element_type=jnp.float32)
    pr1_ref[...] = lax.dot_general(flat, p1_ref[...], (((1,), (1,)), ((), ())),
                                   preferred_element_type=jnp.float32)
    hd2_ref[...] = lax.dot_general(flat, h2_ref[...], (((1,), (1,)), ((), ())),
                                   preferred_element_type=jnp.float32)


def _lstm(x, W_ih, W_hh, b2, tail0_proj, tail1_proj, headW2):
    return pl.pallas_call(
        _lstm_body,
        out_shape=[
            jax.ShapeDtypeStruct((S, B, H), jnp.float32),
            jax.ShapeDtypeStruct((N, H // 2), jnp.float32),
            jax.ShapeDtypeStruct((N, H // 4), jnp.float32),
            jax.ShapeDtypeStruct((N, 2), jnp.float32),
        ],
        scratch_shapes=[pltpu.VMEM((S, B, 4 * H), jnp.float32)],
    )(x, W_ih, W_hh, b2, tail0_proj, tail1_proj, headW2)


def _lse_sum(proj, w, rows, chunk):
    grid = -(-rows // chunk)
    last = rows - (grid - 1) * chunk
    k_dim = proj.shape[1]

    def body(p_ref, w_ref, s_ref):
        i = pl.program_id(0)

        @pl.when(i == 0)
        def _():
            s_ref[...] = jnp.zeros_like(s_ref)

        logits = lax.dot_general(
            p_ref[...].astype(jnp.bfloat16), w_ref[...].astype(jnp.bfloat16),
            (((1,), (1,)), ((), ())),
            preferred_element_type=jnp.float32)
        e = jnp.exp(logits)
        ones = jnp.ones((chunk, 1), jnp.bfloat16)
        if last == chunk:
            s_ref[...] += lax.dot_general(
                e.astype(jnp.bfloat16), ones, (((1,), (0,)), ((), ())),
                preferred_element_type=jnp.float32)
        else:
            @pl.when(i < grid - 1)
            def _():
                s_ref[...] += lax.dot_general(
                    e.astype(jnp.bfloat16), ones, (((1,), (0,)), ((), ())),
                    preferred_element_type=jnp.float32)

            @pl.when(i == grid - 1)
            def _():
                col = lax.broadcasted_iota(jnp.int32, e.shape, 1)
                em = jnp.where(col < last, e, 0.0)
                s_ref[...] += lax.dot_general(
                    em.astype(jnp.bfloat16), ones, (((1,), (0,)), ((), ())),
                    preferred_element_type=jnp.float32)

    return pl.pallas_call(
        body,
        grid=(grid,),
        in_specs=[
            pl.BlockSpec((N, k_dim), lambda i: (0, 0)),
            pl.BlockSpec((chunk, k_dim), lambda i: (i, 0)),
        ],
        out_specs=pl.BlockSpec((N, 1), lambda i: (0, 0)),
        out_shape=jax.ShapeDtypeStruct((N, 1), jnp.float32),
        compiler_params=pltpu.CompilerParams(
            dimension_semantics=("arbitrary",)),
    )(proj, w)


def _assemble_body(tgt_ref, sh_ref, s0_ref, s1_ref, hd2_ref, fl_ref, wh_ref,
                   p0_ref, w0_ref, p1_ref, w1_ref, o_ref):
    tgt = tgt_ref[...]
    lse_h = jnp.log(sh_ref[...])
    lse0 = jnp.log(s0_ref[...])
    lse1 = jnp.log(s1_ref[...])
    th = jnp.sum(fl_ref[...] * wh_ref[...], axis=1, keepdims=True)
    t0 = jnp.sum(p0_ref[...] * w0_ref[...], axis=1, keepdims=True)
    t1 = jnp.sum(p1_ref[...] * w1_ref[...], axis=1, keepdims=True)
    hd2 = hd2_ref[...]
    out = jnp.where(tgt < C0, th - lse_h, 0.0)
    out = jnp.where((tgt >= C0) & (tgt < C1),
                    hd2[:, 0:1] - lse_h + t0 - lse0, out)
    out = jnp.where(tgt >= C1, hd2[:, 1:2] - lse_h + t1 - lse1, out)
    o_ref[...] = jnp.full((1, 1), -1.0 / N, jnp.float32) * jnp.sum(out)


def _assemble(tgt2, s_h, s_0, s_1, hd2, flat, wh, pr0, w0, pr1, w1):
    return pl.pallas_call(
        _assemble_body,
        out_shape=jax.ShapeDtypeStruct((1, 1), jnp.float32),
    )(tgt2, s_h, s_0, s_1, hd2, flat, wh, pr0, w0, pr1, w1)


def kernel(review_input, review_output, emb, W_ih, W_hh, b_ih, b_hh,
           head_W, tail0_proj, tail0_out, tail1_proj, tail1_out):
    ie = review_input.reshape(-1).astype(jnp.int32)
    tgt = review_output.reshape(-1).astype(jnp.int32)
    ih = jnp.clip(tgt, 0, C0 - 1)
    i0 = jnp.clip(tgt - C0, 0, C1 - C0 - 1)
    i1 = jnp.clip(tgt - C1, 0, V - C1 - 1)

    x = _row_gather(emb, ie, D, "emb_row_gather")
    wh = _gather_head(head_W, ih)
    w0 = _row_gather(tail0_out, i0, H // 2, "tail0_row_gather")
    w1 = _row_gather(tail1_out, i1, H // 4, "tail1_row_gather")

    b2 = (b_ih + b_hh).reshape(1, 4 * H)
    headW2 = lax.slice(head_W, (C0, 0), (C0 + 2, H))
    flat3, pr0, pr1, hd2 = _lstm(x, W_ih, W_hh, b2, tail0_proj,
                                 tail1_proj, headW2)
    flat = flat3.reshape(N, H)

    s_h = _lse_sum(flat, head_W, HEAD, 2048)
    s_0 = _lse_sum(pr0, tail0_out, C1 - C0, 2048)
    s_1 = _lse_sum(pr1, tail1_out, V - C1, 2048)

    loss = _assemble(tgt.reshape(N, 1), s_h, s_0, s_1, hd2, flat, wh,
                     pr0, w0, pr1, w1)
    return loss.reshape(())


# speedup vs baseline: 1.3360x; 1.2415x over previous
import functools

import jax
import jax.numpy as jnp
from jax import lax
from jax.experimental import pallas as pl
from jax.experimental.pallas import tpu as pltpu
from jax.experimental.pallas import tpu_sc as plsc

V = 100000
D = 64
H = 128
C0 = 6667
C1 = 20001
S = 50
B = 32
HEAD = C0 + 2
N = S * B

_SC_ROWS = 64
_SC_WORKERS = N // _SC_ROWS


def _row_gather(tbl, idx, width, label):
    mesh = plsc.VectorSubcoreMesh(core_axis_name="c", subcore_axis_name="s")

    @functools.partial(
        pl.kernel,
        mesh=mesh,
        out_type=jax.ShapeDtypeStruct((N, width), jnp.float32),
        scratch_types=[
            pltpu.VMEM((_SC_ROWS,), jnp.int32),
            pltpu.VMEM((_SC_ROWS, width), jnp.float32),
            pltpu.SemaphoreType.DMA,
        ],
        name=label,
        compiler_params=pltpu.CompilerParams(needs_layout_passes=False),
    )
    def gather_kernel(tbl_h, idx_h, out_o, idx_v, rows_v, sem):
        wid = lax.axis_index("s") * 2 + lax.axis_index("c")

        @pl.when(wid < _SC_WORKERS)
        def _():
            base = wid * _SC_ROWS
            pltpu.sync_copy(idx_h.at[pl.ds(base, _SC_ROWS)], idx_v)
            lane = lax.broadcasted_iota(jnp.int32, (16,), 0)
            for g in range(_SC_ROWS // 16):
                grp = idx_v[pl.ds(g * 16, 16)]
                for l in range(16):
                    r = jnp.sum(jnp.where(lane == l, grp, 0))
                    pltpu.async_copy(tbl_h.at[pl.ds(r, 1), :],
                                     rows_v.at[pl.ds(g * 16 + l, 1), :], sem)
            pltpu.make_async_copy(tbl_h.at[pl.ds(0, _SC_ROWS), :],
                                  rows_v, sem).wait()
            pltpu.sync_copy(rows_v, out_o.at[pl.ds(base, _SC_ROWS)])

    return gather_kernel(tbl, idx)


def _gather_head(head_W, idx_h):
    mesh = plsc.VectorSubcoreMesh(core_axis_name="c", subcore_axis_name="s")

    @functools.partial(
        pl.kernel,
        mesh=mesh,
        out_type=jax.ShapeDtypeStruct((N, H), jnp.float32),
        scratch_types=[
            pltpu.VMEM((_SC_ROWS,), jnp.int32),
            pltpu.VMEM((_SC_ROWS, H), jnp.float32),
            pltpu.SemaphoreType.DMA,
        ],
    )
    def gather_kernel(hw_h, ih_h, wh_o, ih_v, wh_v, sem):
        wid = lax.axis_index("s") * 2 + lax.axis_index("c")

        @pl.when(wid < _SC_WORKERS)
        def _():
            base = wid * _SC_ROWS
            pltpu.sync_copy(ih_h.at[pl.ds(base, _SC_ROWS)], ih_v)
            pltpu.async_copy(hw_h.at[ih_v], wh_v, sem).wait()
            pltpu.sync_copy(wh_v, wh_o.at[pl.ds(base, _SC_ROWS)])

    return gather_kernel(head_W, idx_h)


def _lstm_body(x_ref, wih_ref, whh_ref, b_ref, p0_ref, p1_ref,
               h2_ref, flat_ref, pr0_ref, pr1_ref, hd2_ref, xp_ref):
    x = x_ref[...]
    xp = lax.dot_general(
        x.astype(jnp.bfloat16), wih_ref[...].astype(jnp.bfloat16),
        (((1,), (1,)), ((), ())),
        preferred_element_type=jnp.float32) + b_ref[...]
    xp_ref[...] = xp.reshape(S, B, 4 * H)
    whh_b = whh_ref[...].astype(jnp.bfloat16)

    def step(t, carry):
        h, c = carry
        z = xp_ref[t] + lax.dot_general(
            h.astype(jnp.bfloat16), whh_b, (((1,), (1,)), ((), ())),
            preferred_element_type=jnp.float32)
        i = jax.nn.sigmoid(z[:, :H])
        f = jax.nn.sigmoid(z[:, H:2 * H])
        g = jnp.tanh(z[:, 2 * H:3 * H])
        o = jax.nn.sigmoid(z[:, 3 * H:])
        c = f * c + i * g
        h = o * jnp.tanh(c)
        flat_ref[t] = h
        return (h, c)

    lax.fori_loop(0, S, step,
                  (jnp.zeros((B, H), jnp.float32),
                   jnp.zeros((B, H), jnp.float32)),
                  unroll=2)
    flat = flat_ref[...].reshape(N, H)
    pr0_ref[...] = lax.dot_general(flat, p0_ref[...], (((1,), (1,)), ((), ())),
                                   preferred_element_type=jnp.float32)
    pr1_ref[...] = lax.dot_general(flat, p1_ref[...], (((1,), (1,)), ((), ())),
                                   preferred_element_type=jnp.float32)
    hd2_ref[...] = lax.dot_general(flat, h2_ref[...], (((1,), (1,)), ((), ())),
                                   preferred_element_type=jnp.float32)


def _lstm(x, W_ih, W_hh, b2, tail0_proj, tail1_proj, headW2):
    return pl.pallas_call(
        _lstm_body,
        out_shape=[
            jax.ShapeDtypeStruct((S, B, H), jnp.float32),
            jax.ShapeDtypeStruct((N, H // 2), jnp.float32),
            jax.ShapeDtypeStruct((N, H // 4), jnp.float32),
            jax.ShapeDtypeStruct((N, 2), jnp.float32),
        ],
        scratch_shapes=[pltpu.VMEM((S, B, 4 * H), jnp.float32)],
    )(x, W_ih, W_hh, b2, tail0_proj, tail1_proj, headW2)


def _lse_sum(proj, w, rows, chunk):
    grid = -(-rows // chunk)
    last = rows - (grid - 1) * chunk
    k_dim = proj.shape[1]

    def body(p_ref, w_ref, s_ref):
        i = pl.program_id(0)

        @pl.when(i == 0)
        def _():
            s_ref[...] = jnp.zeros_like(s_ref)

        logits = lax.dot_general(
            p_ref[...].astype(jnp.bfloat16), w_ref[...].astype(jnp.bfloat16),
            (((1,), (1,)), ((), ())),
            preferred_element_type=jnp.float32)
        e = jnp.exp(logits)
        if last == chunk:
            s_ref[...] += jnp.sum(e, axis=1, keepdims=True)
        else:
            @pl.when(i < grid - 1)
            def _():
                s_ref[...] += jnp.sum(e, axis=1, keepdims=True)

            @pl.when(i == grid - 1)
            def _():
                col = lax.broadcasted_iota(jnp.int32, e.shape, 1)
                s_ref[...] += jnp.sum(jnp.where(col < last, e, 0.0),
                                      axis=1, keepdims=True)

    return pl.pallas_call(
        body,
        grid=(grid,),
        in_specs=[
            pl.BlockSpec((N, k_dim), lambda i: (0, 0)),
            pl.BlockSpec((chunk, k_dim), lambda i: (i, 0)),
        ],
        out_specs=pl.BlockSpec((N, 1), lambda i: (0, 0)),
        out_shape=jax.ShapeDtypeStruct((N, 1), jnp.float32),
        compiler_params=pltpu.CompilerParams(
            dimension_semantics=("arbitrary",)),
    )(proj, w)


def _assemble_body(tgt_ref, sh_ref, s0_ref, s1_ref, hd2_ref, fl_ref, wh_ref,
                   p0_ref, w0_ref, p1_ref, w1_ref, o_ref):
    tgt = tgt_ref[...]
    lse_h = jnp.log(sh_ref[...])
    lse0 = jnp.log(s0_ref[...])
    lse1 = jnp.log(s1_ref[...])
    th = jnp.sum(fl_ref[...] * wh_ref[...], axis=1, keepdims=True)
    t0 = jnp.sum(p0_ref[...] * w0_ref[...], axis=1, keepdims=True)
    t1 = jnp.sum(p1_ref[...] * w1_ref[...], axis=1, keepdims=True)
    hd2 = hd2_ref[...]
    out = jnp.where(tgt < C0, th - lse_h, 0.0)
    out = jnp.where((tgt >= C0) & (tgt < C1),
                    hd2[:, 0:1] - lse_h + t0 - lse0, out)
    out = jnp.where(tgt >= C1, hd2[:, 1:2] - lse_h + t1 - lse1, out)
    o_ref[...] = jnp.full((1, 1), -1.0 / N, jnp.float32) * jnp.sum(out)


def _assemble(tgt2, s_h, s_0, s_1, hd2, flat, wh, pr0, w0, pr1, w1):
    return pl.pallas_call(
        _assemble_body,
        out_shape=jax.ShapeDtypeStruct((1, 1), jnp.float32),
    )(tgt2, s_h, s_0, s_1, hd2, flat, wh, pr0, w0, pr1, w1)


def kernel(review_input, review_output, emb, W_ih, W_hh, b_ih, b_hh,
           head_W, tail0_proj, tail0_out, tail1_proj, tail1_out):
    ie = review_input.reshape(-1).astype(jnp.int32)
    tgt = review_output.reshape(-1).astype(jnp.int32)
    ih = jnp.clip(tgt, 0, C0 - 1)
    i0 = jnp.clip(tgt - C0, 0, C1 - C0 - 1)
    i1 = jnp.clip(tgt - C1, 0, V - C1 - 1)

    x = _row_gather(emb, ie, D, "emb_row_gather")
    wh = _gather_head(head_W, ih)
    w0 = _row_gather(tail0_out, i0, H // 2, "tail0_row_gather")
    w1 = _row_gather(tail1_out, i1, H // 4, "tail1_row_gather")

    b2 = (b_ih + b_hh).reshape(1, 4 * H)
    headW2 = lax.slice(head_W, (C0, 0), (C0 + 2, H))
    flat3, pr0, pr1, hd2 = _lstm(x, W_ih, W_hh, b2, tail0_proj,
                                 tail1_proj, headW2)
    flat = flat3.reshape(N, H)

    s_h = _lse_sum(flat, head_W, HEAD, 4096)
    s_0 = _lse_sum(pr0, tail0_out, C1 - C0, 4096)
    s_1 = _lse_sum(pr1, tail1_out, V - C1, 4096)

    loss = _assemble(tgt.reshape(N, 1), s_h, s_0, s_1, hd2, flat, wh,
                     pr0, w0, pr1, w1)
    return loss.reshape(())
